# baseline (speedup 1.0000x reference)
import functools

import jax
import jax.numpy as jnp
from jax import lax
from jax.experimental import pallas as pl
from jax.experimental.pallas import tpu as pltpu
from jax.experimental.pallas import tpu_sc as plsc

_B, _W = 16384, 100
_NC, _NS, _L = 2, 16, 16
_NW = _NC * _NS
_RPW = _B // _NW
_CHUNKS = 4
_CROWS = _RPW // _CHUNKS
_CGROUPS = _CROWS // _L


def _make_sc_call():
    mesh = plsc.VectorSubcoreMesh(
        core_axis_name="c", subcore_axis_name="s",
        num_cores=_NC, num_subcores=_NS)

    @functools.partial(
        pl.kernel,
        mesh=mesh,
        compiler_params=pltpu.CompilerParams(needs_layout_passes=False),
        out_type=jax.ShapeDtypeStruct((_NW, _L), jnp.float32),
        scratch_types=[
            pltpu.VMEM((_CROWS, _W), jnp.float32),
            pltpu.VMEM((_CROWS, _W), jnp.float32),
            pltpu.VMEM((_CROWS, _W), jnp.float32),
            pltpu.VMEM((_CROWS, _W), jnp.float32),
            pltpu.VMEM((_W, _W), jnp.float32),
            pltpu.VMEM((_L,), jnp.float32),
            pltpu.SemaphoreType.DMA,
            pltpu.SemaphoreType.DMA,
            pltpu.SemaphoreType.DMA,
            pltpu.SemaphoreType.DMA,
            pltpu.SemaphoreType.DMA,
        ],
    )
    def sc_loss(predict_hbm, target_hbm, pm_hbm, out_hbm,
                pred0, pred1, targ0, targ1, pm_v, acc_v,
                sp0, sp1, st0, st1, spm):
        preds, targs = (pred0, pred1), (targ0, targ1)
        psems, tsems = (sp0, sp1), (st0, st1)
        wid = lax.axis_index("s") * _NC + lax.axis_index("c")
        base = wid * _RPW

        handles = {}

        def fire(c):
            slot = c % 2
            row0 = base + c * _CROWS
            handles[c] = (
                pltpu.async_copy(predict_hbm.at[pl.ds(row0, _CROWS)],
                                 preds[slot], psems[slot]),
                pltpu.async_copy(target_hbm.at[pl.ds(row0, _CROWS)],
                                 targs[slot], tsems[slot]),
            )

        pm_h = pltpu.async_copy(pm_hbm, pm_v, spm)
        fire(0)
        fire(1)
        pm_h.wait()

        lanes = lax.iota(jnp.int32, _L)
        zero_f = jnp.zeros((_L,), jnp.float32)
        zero_i = jnp.zeros((_L,), jnp.int32)
        neg_inf = jnp.full((_L,), -jnp.inf, jnp.float32)

        def make_group_body(pred_v, targ_v):
            def group_body(g, acc):
                rows = g * _L + lanes

                m, a, s = neg_inf, zero_i, zero_f
                tm, t = neg_inf, zero_i
                for j in range(_W):
                    jv = jnp.full((_L,), j, jnp.int32)
                    p = plsc.load_gather(pred_v, [rows, jv])
                    s = s + jnp.exp(p)
                    upd = p > m
                    m = jnp.where(upd, p, m)
                    a = jnp.where(upd, jv, a)
                    q = plsc.load_gather(targ_v, [rows, jv])
                    upd2 = q > tm
                    tm = jnp.where(upd2, q, tm)
                    t = jnp.where(upd2, jv, t)

                pm_val = plsc.load_gather(pm_v, [t, a])
                contrib = jnp.where(a != t, pm_val * jnp.exp(m) / s, zero_f)
                return acc + contrib
            return group_body

        acc = zero_f
        for c in range(_CHUNKS):
            h0, h1 = handles[c]
            h0.wait()
            h1.wait()
            slot = c % 2
            acc = lax.fori_loop(0, _CGROUPS,
                                make_group_body(preds[slot], targs[slot]), acc)
            if c + 2 < _CHUNKS:
                fire(c + 2)

        acc_v[...] = acc
        pltpu.sync_copy(acc_v, out_hbm.at[wid])

    return sc_loss


_SC_LOSS_CACHE = []


def kernel(predict, target, penalty_matrix):
    if not _SC_LOSS_CACHE:
        _SC_LOSS_CACHE.append(jax.jit(_make_sc_call()))
    partials = _SC_LOSS_CACHE[0](predict, target, penalty_matrix)
    return jnp.sum(partials) / jnp.float32(predict.shape[0])


# speedup vs baseline: 1.3848x; 1.3848x over previous
import functools

import jax
import jax.numpy as jnp
from jax import lax
from jax.experimental import pallas as pl
from jax.experimental.pallas import tpu as pltpu
from jax.experimental.pallas import tpu_sc as plsc

_B, _W = 16384, 100
_NC, _NS, _L = 2, 16, 16
_NW = _NC * _NS
_RPW = _B // _NW

_TC_GRID = 16
_TC_ROWS = _B // _TC_GRID
_OUT_R = 128


def _tc_stats_body(pred_ref, targ_ref, a_ref, t_ref, val_ref):
    x = pred_ref[...]
    y = targ_ref[...]
    iot = lax.broadcasted_iota(jnp.int32, (_TC_ROWS, _W), 1)
    big = jnp.int32(_W)

    m = jnp.max(x, axis=1)
    a = jnp.min(jnp.where(x == m[:, None], iot, big), axis=1)
    s = jnp.sum(jnp.exp(x), axis=1)
    tm = jnp.max(y, axis=1)
    t = jnp.min(jnp.where(y == tm[:, None], iot, big), axis=1)
    val = jnp.where(a != t, jnp.exp(m) / s, jnp.float32(0.0))

    rows = _TC_ROWS // _OUT_R
    a_ref[...] = a.reshape(rows, _OUT_R)
    t_ref[...] = t.reshape(rows, _OUT_R)
    val_ref[...] = val.reshape(rows, _OUT_R)


def _make_tc_stats():
    rows = _TC_ROWS // _OUT_R
    return pl.pallas_call(
        _tc_stats_body,
        grid=(_TC_GRID,),
        in_specs=[
            pl.BlockSpec((_TC_ROWS, _W), lambda g: (g, 0)),
            pl.BlockSpec((_TC_ROWS, _W), lambda g: (g, 0)),
        ],
        out_specs=[
            pl.BlockSpec((rows, _OUT_R), lambda g: (g, 0)),
            pl.BlockSpec((rows, _OUT_R), lambda g: (g, 0)),
            pl.BlockSpec((rows, _OUT_R), lambda g: (g, 0)),
        ],
        out_shape=[
            jax.ShapeDtypeStruct((_OUT_R, _OUT_R), jnp.int32),
            jax.ShapeDtypeStruct((_OUT_R, _OUT_R), jnp.int32),
            jax.ShapeDtypeStruct((_OUT_R, _OUT_R), jnp.float32),
        ],
    )


_SLAB_R = _RPW // _OUT_R


def _make_sc_gather():
    mesh = plsc.VectorSubcoreMesh(
        core_axis_name="c", subcore_axis_name="s",
        num_cores=_NC, num_subcores=_NS)

    @functools.partial(
        pl.kernel,
        mesh=mesh,
        compiler_params=pltpu.CompilerParams(needs_layout_passes=False),
        out_type=jax.ShapeDtypeStruct((_NW, _L), jnp.float32),
        scratch_types=[
            pltpu.VMEM((_SLAB_R, _OUT_R), jnp.int32),
            pltpu.VMEM((_SLAB_R, _OUT_R), jnp.int32),
            pltpu.VMEM((_SLAB_R, _OUT_R), jnp.float32),
            pltpu.VMEM((_W, _W), jnp.float32),
            pltpu.VMEM((_L,), jnp.float32),
        ],
    )
    def sc_gather(a_hbm, t_hbm, val_hbm, pm_hbm, out_hbm,
                  a_v, t_v, val_v, pm_v, acc_v):
        wid = lax.axis_index("s") * _NC + lax.axis_index("c")
        r0 = wid * _SLAB_R
        pltpu.sync_copy(a_hbm.at[pl.ds(r0, _SLAB_R)], a_v)
        pltpu.sync_copy(t_hbm.at[pl.ds(r0, _SLAB_R)], t_v)
        pltpu.sync_copy(val_hbm.at[pl.ds(r0, _SLAB_R)], val_v)
        pltpu.sync_copy(pm_hbm, pm_v)

        acc = jnp.zeros((_L,), jnp.float32)
        for r in range(_SLAB_R):
            for cb in range(_OUT_R // _L):
                c = cb * _L
                av = a_v[r, pl.ds(c, _L)]
                tv = t_v[r, pl.ds(c, _L)]
                vv = val_v[r, pl.ds(c, _L)]
                pmv = plsc.load_gather(pm_v, [tv, av])
                acc = acc + pmv * vv
        acc_v[...] = acc
        pltpu.sync_copy(acc_v, out_hbm.at[wid])

    return sc_gather


_CALLS = {}


def kernel(predict, target, penalty_matrix):
    if not _CALLS:
        _CALLS["tc"] = _make_tc_stats()
        _CALLS["sc"] = jax.jit(_make_sc_gather())
    a, t, val = _CALLS["tc"](predict, target)
    partials = _CALLS["sc"](a, t, val, penalty_matrix)
    return jnp.sum(partials) / jnp.float32(predict.shape[0])
